# baseline (speedup 1.0000x reference)
import functools

import jax
import jax.numpy as jnp
from jax import lax
from jax.experimental import pallas as pl
from jax.experimental.pallas import tpu as pltpu
from jax.experimental.pallas import tpu_sc as plsc

B = 16384
UD = 16
CD = 8
ND = 8
H1 = 64
H2 = 32

_NC = 2
_NS = 16
_NW = _NC * _NS
_BPW = B // _NW
_CHUNK = 128
_NCH = _BPW // _CHUNK


def _make_gather():
    mesh = plsc.VectorSubcoreMesh(core_axis_name="c", subcore_axis_name="s")

    @functools.partial(
        pl.kernel,
        mesh=mesh,
        out_type=[
            jax.ShapeDtypeStruct((UD, B), jnp.float32),
            jax.ShapeDtypeStruct((CD, B), jnp.float32),
        ],
        scratch_types=[
            pltpu.VMEM((_NCH, _CHUNK), jnp.int32),
            pltpu.VMEM((_NCH, _CHUNK), jnp.int32),
            pltpu.VMEM((UD, _BPW), jnp.float32),
            pltpu.VMEM((CD, _BPW), jnp.float32),
            pltpu.SemaphoreType.DMA,
        ],
        compiler_params=pltpu.CompilerParams(use_tc_tiling_on_sc=False),
    )
    def gather_k(uid_hbm, cid_hbm, utab_hbm, ctab_hbm, u_out, c_out,
                 uidx_v, cidx_v, urows_v, crows_v, sem):
        wid = lax.axis_index("s") * _NC + lax.axis_index("c")
        base = wid * _BPW
        for j in range(_NCH):
            pltpu.sync_copy(uid_hbm.at[pl.ds(base + j * _CHUNK, _CHUNK)],
                            uidx_v.at[j])
            pltpu.sync_copy(cid_hbm.at[pl.ds(base + j * _CHUNK, _CHUNK)],
                            cidx_v.at[j])
        copies = []
        for f in range(UD):
            for j in range(_NCH):
                copies.append(pltpu.async_copy(
                    utab_hbm.at[f].at[uidx_v.at[j]],
                    urows_v.at[f, pl.ds(j * _CHUNK, _CHUNK)], sem))
        for f in range(CD):
            for j in range(_NCH):
                copies.append(pltpu.async_copy(
                    ctab_hbm.at[f].at[cidx_v.at[j]],
                    crows_v.at[f, pl.ds(j * _CHUNK, _CHUNK)], sem))
        for cp in copies:
            cp.wait()
        pltpu.sync_copy(urows_v, u_out.at[:, pl.ds(base, _BPW)])
        pltpu.sync_copy(crows_v, c_out.at[:, pl.ds(base, _BPW)])

    return gather_k


_gather_cache = []


def _gather(*args):
    if not _gather_cache:
        _gather_cache.append(_make_gather())
    return _gather_cache[0](*args)


def _mlp_body(u_ref, c_ref, n_ref, w1u_ref, w1c_ref, w1n_ref, b1_ref,
              w2_ref, b2_ref, w3_ref, b3_ref, out_ref):
    h = (jnp.dot(w1u_ref[...], u_ref[...], preferred_element_type=jnp.float32)
         + jnp.dot(w1c_ref[...], c_ref[...], preferred_element_type=jnp.float32)
         + jnp.dot(w1n_ref[...], n_ref[...], preferred_element_type=jnp.float32)
         + b1_ref[...])
    h = jnp.maximum(h, 0.0)
    h = jnp.maximum(
        jnp.dot(w2_ref[...], h, preferred_element_type=jnp.float32) + b2_ref[...],
        0.0)
    out_ref[...] = (jnp.dot(w3_ref[...], h, preferred_element_type=jnp.float32)
                    + b3_ref[...])


def kernel(user_id, cat_id, numeric, user_table, cat_table, W1, b1, W2, b2, W3, b3):
    u_t, c_t = _gather(user_id.astype(jnp.int32), cat_id.astype(jnp.int32),
                       user_table.T, cat_table.T)
    out = pl.pallas_call(
        _mlp_body,
        out_shape=jax.ShapeDtypeStruct((1, B), jnp.float32),
    )(u_t, c_t, numeric.T, W1[:, :UD], W1[:, UD:UD + CD], W1[:, UD + CD:],
      b1.reshape(H1, 1), W2, b2.reshape(H2, 1), W3, b3.reshape(1, 1))
    return out.reshape(B)


# speedup vs baseline: 1.3020x; 1.3020x over previous
import functools

import jax
import jax.numpy as jnp
from jax import lax
from jax.experimental import pallas as pl
from jax.experimental.pallas import tpu as pltpu
from jax.experimental.pallas import tpu_sc as plsc

B = 16384
NU = 1000000
NCAT = 1000
UD = 16
CD = 8
ND = 8
H1 = 64
H2 = 32

_NC = 2
_NS = 16
_NW = _NC * _NS
_BPW = B // _NW
_CHUNK = 128
_NCH = _BPW // _CHUNK

_PACK = 128 // UD
_NROWS = NU // _PACK
_PCHUNK = NU // 8


_SLAB = 16384
_NSLAB = NU // _SLAB
_TAIL = 64
_MID = NU - _NSLAB * _SLAB - _TAIL


def _pack_body(in_hbm, tail_ref, out_hbm, in_v, out_v, sem_in, sem_out):
    def step(i, carry):
        pltpu.async_copy(in_hbm.at[:, pl.ds(i * _SLAB, _SLAB)], in_v,
                         sem_in).wait()
        out_v[...] = pltpu.einshape("a(bc)->b(ca)", in_v[...], c=_PACK)
        pltpu.async_copy(out_v, out_hbm.at[pl.ds(i * (_SLAB // _PACK),
                                                 _SLAB // _PACK), :],
                         sem_out).wait()
        return carry

    lax.fori_loop(0, _NSLAB, step, 0)
    pltpu.async_copy(in_hbm.at[:, pl.ds(_NSLAB * _SLAB, _MID)],
                     in_v.at[:, pl.ds(0, _MID)], sem_in).wait()
    out_v[pl.ds(0, _MID // _PACK), :] = pltpu.einshape(
        "a(bc)->b(ca)", in_v[:, pl.ds(0, _MID)], c=_PACK)
    pltpu.async_copy(out_v.at[pl.ds(0, _MID // _PACK)],
                     out_hbm.at[pl.ds(_NSLAB * (_SLAB // _PACK),
                                      _MID // _PACK), :],
                     sem_out).wait()
    out_v[pl.ds(0, _TAIL // _PACK), :] = pltpu.einshape(
        "a(bc)->b(ca)", tail_ref[...], c=_PACK)
    pltpu.async_copy(out_v.at[pl.ds(0, _TAIL // _PACK)],
                     out_hbm.at[pl.ds(_NROWS - _TAIL // _PACK,
                                      _TAIL // _PACK), :],
                     sem_out).wait()


def _pack_table(ut_t, ut_tail):
    return pl.pallas_call(
        _pack_body,
        in_specs=[pl.BlockSpec(memory_space=pl.ANY),
                  pl.BlockSpec(memory_space=pltpu.VMEM)],
        out_specs=pl.BlockSpec(memory_space=pl.ANY),
        out_shape=jax.ShapeDtypeStruct((_NROWS, 128), jnp.float32),
        scratch_shapes=[
            pltpu.VMEM((UD, _SLAB), jnp.float32),
            pltpu.VMEM((_SLAB // _PACK, 128), jnp.float32),
            pltpu.SemaphoreType.DMA,
            pltpu.SemaphoreType.DMA,
        ],
    )(ut_t, ut_tail)


def _make_user_gather():
    mesh = plsc.VectorSubcoreMesh(core_axis_name="c", subcore_axis_name="s")

    @functools.partial(
        pl.kernel,
        mesh=mesh,
        out_type=jax.ShapeDtypeStruct((UD, B), jnp.float32),
        scratch_types=[
            pltpu.VMEM((_NCH, _CHUNK), jnp.int32),
            pltpu.VMEM((_NCH, _CHUNK), jnp.int32),
            pltpu.VMEM((_BPW, 128), jnp.float32),
            pltpu.VMEM((UD, _BPW), jnp.float32),
            pltpu.SemaphoreType.DMA,
        ],
        compiler_params=pltpu.CompilerParams(needs_layout_passes=False),
    )
    def user_k(uid_hbm, ut2_hbm, u_out, idx_v, ridx_v, rows_v, out_v, sem):
        wid = lax.axis_index("s") * _NC + lax.axis_index("c")
        base = wid * _BPW
        for j in range(_NCH):
            pltpu.sync_copy(uid_hbm.at[pl.ds(base + j * _CHUNK, _CHUNK)],
                            idx_v.at[j])
        for j in range(_NCH):
            for t in range(_CHUNK // 16):
                v = idx_v[j, pl.ds(t * 16, 16)]
                ridx_v[j, pl.ds(t * 16, 16)] = lax.shift_right_logical(v, 3)
        copies = []
        for j in range(_NCH):
            copies.append(pltpu.async_copy(
                ut2_hbm.at[ridx_v.at[j]],
                rows_v.at[pl.ds(j * _CHUNK, _CHUNK)], sem))
        for cp in copies:
            cp.wait()

        lane = lax.iota(jnp.int32, 16)

        def extract(g, carry):
            j = g // (_CHUNK // 16)
            t = g % (_CHUNK // 16)
            ids = idx_v[j, pl.ds(t * 16, 16)]
            col0 = (ids & 7) * 16
            rows = g * 16 + lane
            for f in range(UD):
                vals = plsc.load_gather(rows_v, [rows, col0 + f])
                out_v[f, pl.ds(g * 16, 16)] = vals
            return carry

        lax.fori_loop(0, _BPW // 16, extract, 0, unroll=False)
        pltpu.sync_copy(out_v, u_out.at[:, pl.ds(base, _BPW)])

    return user_k


def _make_cat_gather():
    mesh = plsc.VectorSubcoreMesh(core_axis_name="c", subcore_axis_name="s")

    @functools.partial(
        pl.kernel,
        mesh=mesh,
        out_type=jax.ShapeDtypeStruct((CD, B), jnp.float32),
        scratch_types=[
            pltpu.VMEM((_NCH, _CHUNK), jnp.int32),
            pltpu.VMEM((CD, _BPW), jnp.float32),
            pltpu.SemaphoreType.DMA,
        ],
        compiler_params=pltpu.CompilerParams(use_tc_tiling_on_sc=False),
    )
    def cat_k(cid_hbm, ctab_hbm, c_out, cidx_v, crows_v, sem):
        wid = lax.axis_index("s") * _NC + lax.axis_index("c")
        base = wid * _BPW
        for j in range(_NCH):
            pltpu.sync_copy(cid_hbm.at[pl.ds(base + j * _CHUNK, _CHUNK)],
                            cidx_v.at[j])
        copies = []
        for f in range(CD):
            for j in range(_NCH):
                copies.append(pltpu.async_copy(
                    ctab_hbm.at[f].at[cidx_v.at[j]],
                    crows_v.at[f, pl.ds(j * _CHUNK, _CHUNK)], sem))
        for cp in copies:
            cp.wait()
        pltpu.sync_copy(crows_v, c_out.at[:, pl.ds(base, _BPW)])

    return cat_k


_kernel_cache = {}


def _get(name, maker):
    if name not in _kernel_cache:
        _kernel_cache[name] = maker()
    return _kernel_cache[name]


def _mlp_body(u_ref, c_ref, n_ref, w1u_ref, w1c_ref, w1n_ref, b1_ref,
              w2_ref, b2_ref, w3_ref, b3_ref, out_ref):
    h = (jnp.dot(w1u_ref[...], u_ref[...], preferred_element_type=jnp.float32)
         + jnp.dot(w1c_ref[...], c_ref[...], preferred_element_type=jnp.float32)
         + jnp.dot(w1n_ref[...], n_ref[...], preferred_element_type=jnp.float32)
         + b1_ref[...])
    h = jnp.maximum(h, 0.0)
    h = jnp.maximum(
        jnp.dot(w2_ref[...], h, preferred_element_type=jnp.float32) + b2_ref[...],
        0.0)
    out_ref[...] = (jnp.dot(w3_ref[...], h, preferred_element_type=jnp.float32)
                    + b3_ref[...])


def kernel(user_id, cat_id, numeric, user_table, cat_table, W1, b1, W2, b2, W3, b3):
    ut_t = user_table.T
    ut2 = _pack_table(ut_t, lax.slice(ut_t, (0, NU - _TAIL), (UD, NU)))
    u_t = _get("user", _make_user_gather)(user_id.astype(jnp.int32), ut2)
    c_t = _get("cat", _make_cat_gather)(cat_id.astype(jnp.int32), cat_table.T)
    out = pl.pallas_call(
        _mlp_body,
        out_shape=jax.ShapeDtypeStruct((1, B), jnp.float32),
    )(u_t, c_t, numeric.T, W1[:, :UD], W1[:, UD:UD + CD], W1[:, UD + CD:],
      b1.reshape(H1, 1), W2, b2.reshape(H2, 1), W3, b3.reshape(1, 1))
    return out.reshape(B)


# speedup vs baseline: 5.5510x; 4.2634x over previous
import functools

import jax
import jax.numpy as jnp
from jax import lax
from jax.experimental import pallas as pl
from jax.experimental.pallas import tpu as pltpu
from jax.experimental.pallas import tpu_sc as plsc

B = 16384
NU = 1000000
NCAT = 1000
UD = 16
CD = 8
ND = 8
H1 = 64
H2 = 32

_NC = 2
_NS = 16
_NW = _NC * _NS
_BPW = B // _NW
_CHUNK = 128
_NCH = _BPW // _CHUNK

_PACK = 128 // UD
_NROWS = NU // _PACK
_PCHUNK = NU // 8


_AC = 2048
_NFULL = 488
_NCHUNKS = _NFULL + 1
_LASTBASE = 997888
_TAIL = 64
_NPW_HI = -(-_NCHUNKS // _NW)
_HI_CUT = _NCHUNKS - (_NPW_HI - 1) * _NW


def _make_pack():
    mesh = plsc.VectorSubcoreMesh(core_axis_name="c", subcore_axis_name="s")

    @functools.partial(
        pl.kernel,
        mesh=mesh,
        out_type=jax.ShapeDtypeStruct((_NROWS, 128), jnp.float32),
        scratch_types=[
            pltpu.VMEM((UD, _AC), jnp.float32),
            pltpu.VMEM((_AC // _PACK, 128), jnp.float32),
            pltpu.VMEM((UD, _TAIL), jnp.float32),
            pltpu.VMEM((_TAIL // _PACK, 128), jnp.float32),
            pltpu.SemaphoreType.DMA,
        ],
        compiler_params=pltpu.CompilerParams(needs_layout_passes=False),
    )
    def pack_k(ut_hbm, tail_hbm, ut2_out, in_v, out_v, tin_v, tout_v, sem):
        wid = lax.axis_index("s") * _NC + lax.axis_index("c")
        lane = lax.iota(jnp.int32, 16)
        rowbase = lax.shift_right_logical(lane, 3)
        cols = [(lane & 7) * UD + a for a in range(UD)]
        nmine = jnp.where(wid < _HI_CUT, _NPW_HI, _NPW_HI - 1)

        def transpose(src, dst, ngroups):
            def grp(g, carry):
                row = rowbase + 2 * g
                for a in range(UD):
                    vals = src[a, pl.ds(g * 16, 16)]
                    plsc.store_scatter(dst, [row, cols[a]], vals)
                return carry

            lax.fori_loop(0, ngroups, grp, 0)

        def chunk(i, carry):
            c = wid + i * _NW
            base_u = pl.multiple_of(
                jnp.where(c == _NCHUNKS - 1, _LASTBASE, c * _AC), 128)
            pltpu.sync_copy(ut_hbm.at[:, pl.ds(base_u, _AC)], in_v)
            transpose(in_v, out_v, _AC // 16)
            pltpu.sync_copy(
                out_v,
                ut2_out.at[pl.ds(pl.multiple_of(base_u // _PACK, 8),
                                 _AC // _PACK), :])
            return carry

        lax.fori_loop(0, nmine, chunk, 0)

        @pl.when(wid == 0)
        def _():
            pltpu.sync_copy(tail_hbm, tin_v)
            transpose(tin_v, tout_v, _TAIL // 16)
            pltpu.sync_copy(tout_v,
                            ut2_out.at[pl.ds(_NROWS - _TAIL // _PACK,
                                             _TAIL // _PACK), :])

    return pack_k


def _pack_table(ut_t, ut_tail):
    return _get("pack", _make_pack)(ut_t, ut_tail)


def _make_user_gather():
    mesh = plsc.VectorSubcoreMesh(core_axis_name="c", subcore_axis_name="s")

    @functools.partial(
        pl.kernel,
        mesh=mesh,
        out_type=jax.ShapeDtypeStruct((UD, B), jnp.float32),
        scratch_types=[
            pltpu.VMEM((_NCH, _CHUNK), jnp.int32),
            pltpu.VMEM((_NCH, _CHUNK), jnp.int32),
            pltpu.VMEM((_BPW, 128), jnp.float32),
            pltpu.VMEM((UD, _BPW), jnp.float32),
            pltpu.SemaphoreType.DMA,
        ],
        compiler_params=pltpu.CompilerParams(needs_layout_passes=False),
    )
    def user_k(uid_hbm, ut2_hbm, u_out, idx_v, ridx_v, rows_v, out_v, sem):
        wid = lax.axis_index("s") * _NC + lax.axis_index("c")
        base = wid * _BPW
        for j in range(_NCH):
            pltpu.sync_copy(uid_hbm.at[pl.ds(base + j * _CHUNK, _CHUNK)],
                            idx_v.at[j])
        for j in range(_NCH):
            for t in range(_CHUNK // 16):
                v = idx_v[j, pl.ds(t * 16, 16)]
                ridx_v[j, pl.ds(t * 16, 16)] = lax.shift_right_logical(v, 3)
        copies = []
        for j in range(_NCH):
            copies.append(pltpu.async_copy(
                ut2_hbm.at[ridx_v.at[j]],
                rows_v.at[pl.ds(j * _CHUNK, _CHUNK)], sem))
        for cp in copies:
            cp.wait()

        lane = lax.iota(jnp.int32, 16)

        def extract(g, carry):
            j = g // (_CHUNK // 16)
            t = g % (_CHUNK // 16)
            ids = idx_v[j, pl.ds(t * 16, 16)]
            col0 = (ids & 7) * 16
            rows = g * 16 + lane
            for f in range(UD):
                vals = plsc.load_gather(rows_v, [rows, col0 + f])
                out_v[f, pl.ds(g * 16, 16)] = vals
            return carry

        lax.fori_loop(0, _BPW // 16, extract, 0, unroll=False)
        pltpu.sync_copy(out_v, u_out.at[:, pl.ds(base, _BPW)])

    return user_k


def _make_cat_gather():
    mesh = plsc.VectorSubcoreMesh(core_axis_name="c", subcore_axis_name="s")

    @functools.partial(
        pl.kernel,
        mesh=mesh,
        out_type=jax.ShapeDtypeStruct((CD, B), jnp.float32),
        scratch_types=[
            pltpu.VMEM((_NCH, _CHUNK), jnp.int32),
            pltpu.VMEM((CD, _BPW), jnp.float32),
            pltpu.SemaphoreType.DMA,
        ],
        compiler_params=pltpu.CompilerParams(use_tc_tiling_on_sc=False),
    )
    def cat_k(cid_hbm, ctab_hbm, c_out, cidx_v, crows_v, sem):
        wid = lax.axis_index("s") * _NC + lax.axis_index("c")
        base = wid * _BPW
        for j in range(_NCH):
            pltpu.sync_copy(cid_hbm.at[pl.ds(base + j * _CHUNK, _CHUNK)],
                            cidx_v.at[j])
        copies = []
        for f in range(CD):
            for j in range(_NCH):
                copies.append(pltpu.async_copy(
                    ctab_hbm.at[f].at[cidx_v.at[j]],
                    crows_v.at[f, pl.ds(j * _CHUNK, _CHUNK)], sem))
        for cp in copies:
            cp.wait()
        pltpu.sync_copy(crows_v, c_out.at[:, pl.ds(base, _BPW)])

    return cat_k


_kernel_cache = {}


def _get(name, maker):
    if name not in _kernel_cache:
        _kernel_cache[name] = maker()
    return _kernel_cache[name]


def _mlp_body(u_ref, c_ref, n_ref, w1u_ref, w1c_ref, w1n_ref, b1_ref,
              w2_ref, b2_ref, w3_ref, b3_ref, out_ref):
    h = (jnp.dot(w1u_ref[...], u_ref[...], preferred_element_type=jnp.float32)
         + jnp.dot(w1c_ref[...], c_ref[...], preferred_element_type=jnp.float32)
         + jnp.dot(w1n_ref[...], n_ref[...], preferred_element_type=jnp.float32)
         + b1_ref[...])
    h = jnp.maximum(h, 0.0)
    h = jnp.maximum(
        jnp.dot(w2_ref[...], h, preferred_element_type=jnp.float32) + b2_ref[...],
        0.0)
    out_ref[...] = (jnp.dot(w3_ref[...], h, preferred_element_type=jnp.float32)
                    + b3_ref[...])


def kernel(user_id, cat_id, numeric, user_table, cat_table, W1, b1, W2, b2, W3, b3):
    ut_t = user_table.T
    ut2 = _pack_table(ut_t, lax.slice(ut_t, (0, NU - _TAIL), (UD, NU)))
    u_t = _get("user", _make_user_gather)(user_id.astype(jnp.int32), ut2)
    c_t = _get("cat", _make_cat_gather)(cat_id.astype(jnp.int32), cat_table.T)
    out = pl.pallas_call(
        _mlp_body,
        out_shape=jax.ShapeDtypeStruct((1, B), jnp.float32),
    )(u_t, c_t, numeric.T, W1[:, :UD], W1[:, UD:UD + CD], W1[:, UD + CD:],
      b1.reshape(H1, 1), W2, b2.reshape(H2, 1), W3, b3.reshape(1, 1))
    return out.reshape(B)


# speedup vs baseline: 5.8326x; 1.0507x over previous
import functools

import jax
import jax.numpy as jnp
from jax import lax
from jax.experimental import pallas as pl
from jax.experimental.pallas import tpu as pltpu
from jax.experimental.pallas import tpu_sc as plsc

B = 16384
NU = 1000000
NCAT = 1000
UD = 16
CD = 8
ND = 8
H1 = 64
H2 = 32

_NC = 2
_NS = 16
_NW = _NC * _NS
_BPW = B // _NW
_CHUNK = 128
_NCH = _BPW // _CHUNK

_PACK = 128 // UD
_NROWS = NU // _PACK
_PCHUNK = NU // 8


_AC = 3072
_NFULL = 325
_NCHUNKS = _NFULL + 1
_LASTBASE = 996864
_TAIL = 64
_NPW_HI = -(-_NCHUNKS // _NW)
_HI_CUT = _NCHUNKS - (_NPW_HI - 1) * _NW
_NCPAD = 1024


def _make_pack():
    mesh = plsc.VectorSubcoreMesh(core_axis_name="c", subcore_axis_name="s")

    @functools.partial(
        pl.kernel,
        mesh=mesh,
        out_type=[jax.ShapeDtypeStruct((_NROWS, 128), jnp.float32),
                  jax.ShapeDtypeStruct((_NCPAD // _PACK, 128), jnp.float32)],
        scratch_types=[
            pltpu.VMEM((UD, _AC), jnp.float32),
            pltpu.VMEM((_AC // _PACK, 128), jnp.float32),
            pltpu.VMEM((UD, _TAIL), jnp.float32),
            pltpu.VMEM((_TAIL // _PACK, 128), jnp.float32),
            pltpu.SemaphoreType.DMA,
        ],
        compiler_params=pltpu.CompilerParams(needs_layout_passes=False),
    )
    def pack_k(ut_hbm, tail_hbm, cpad_hbm, ut2_out, ct2_out,
               in_v, out_v, tin_v, tout_v, sem):
        wid = lax.axis_index("s") * _NC + lax.axis_index("c")
        lane = lax.iota(jnp.int32, 16)
        rowbase = lax.shift_right_logical(lane, 3)
        cols = [(lane & 7) * UD + a for a in range(UD)]
        nmine = jnp.where(wid < _HI_CUT, _NPW_HI, _NPW_HI - 1)

        def transpose(src, dst, ngroups):
            def grp(g, carry):
                row = rowbase + 2 * g
                for a in range(UD):
                    vals = src[a, pl.ds(g * 16, 16)]
                    plsc.store_scatter(dst, [row, cols[a]], vals)
                return carry

            lax.fori_loop(0, ngroups, grp, 0)

        def chunk(i, carry):
            c = wid + i * _NW
            base_u = pl.multiple_of(
                jnp.where(c == _NCHUNKS - 1, _LASTBASE, c * _AC), 128)
            pltpu.sync_copy(ut_hbm.at[:, pl.ds(base_u, _AC)], in_v)
            transpose(in_v, out_v, _AC // 16)
            pltpu.sync_copy(
                out_v,
                ut2_out.at[pl.ds(pl.multiple_of(base_u // _PACK, 8),
                                 _AC // _PACK), :])
            return carry

        lax.fori_loop(0, nmine, chunk, 0)

        @pl.when(wid == 0)
        def _():
            pltpu.sync_copy(tail_hbm, tin_v)
            transpose(tin_v, tout_v, _TAIL // 16)
            pltpu.sync_copy(tout_v,
                            ut2_out.at[pl.ds(_NROWS - _TAIL // _PACK,
                                             _TAIL // _PACK), :])

        @pl.when(wid == _NW - 1)
        def _():
            pltpu.sync_copy(cpad_hbm, in_v.at[:, pl.ds(0, _NCPAD)])
            transpose(in_v, out_v, _NCPAD // 16)
            pltpu.sync_copy(out_v.at[pl.ds(0, _NCPAD // _PACK)], ct2_out)

    return pack_k


def _pack_table(ut_t, ut_tail, cpad):
    return _get("pack", _make_pack)(ut_t, ut_tail, cpad)


def _make_user_gather():
    mesh = plsc.VectorSubcoreMesh(core_axis_name="c", subcore_axis_name="s")

    @functools.partial(
        pl.kernel,
        mesh=mesh,
        out_type=[jax.ShapeDtypeStruct((UD, B), jnp.float32),
                  jax.ShapeDtypeStruct((UD, B), jnp.float32)],
        scratch_types=[
            pltpu.VMEM((_NCH, _CHUNK), jnp.int32),
            pltpu.VMEM((_NCH, _CHUNK), jnp.int32),
            pltpu.VMEM((_BPW, 128), jnp.float32),
            pltpu.VMEM((UD, _BPW), jnp.float32),
            pltpu.SemaphoreType.DMA,
        ],
        compiler_params=pltpu.CompilerParams(needs_layout_passes=False),
    )
    def user_k(uid_hbm, cid_hbm, ut2_hbm, ct2_hbm, u_out, c_out,
               idx_v, ridx_v, rows_v, out_v, sem):
        wid = lax.axis_index("s") * _NC + lax.axis_index("c")
        base = wid * _BPW
        lane = lax.iota(jnp.int32, 16)

        def gather_one(ids_hbm, tab_hbm, dst_out):
            for j in range(_NCH):
                pltpu.sync_copy(ids_hbm.at[pl.ds(base + j * _CHUNK, _CHUNK)],
                                idx_v.at[j])
            for j in range(_NCH):
                for t in range(_CHUNK // 16):
                    v = idx_v[j, pl.ds(t * 16, 16)]
                    ridx_v[j, pl.ds(t * 16, 16)] = lax.shift_right_logical(v, 3)
            copies = []
            for j in range(_NCH):
                copies.append(pltpu.async_copy(
                    tab_hbm.at[ridx_v.at[j]],
                    rows_v.at[pl.ds(j * _CHUNK, _CHUNK)], sem))
            for cp in copies:
                cp.wait()

            def extract(g, carry):
                j = g // (_CHUNK // 16)
                t = g % (_CHUNK // 16)
                ids = idx_v[j, pl.ds(t * 16, 16)]
                col0 = (ids & 7) * 16
                rows = g * 16 + lane
                for f in range(UD):
                    vals = plsc.load_gather(rows_v, [rows, col0 + f])
                    out_v[f, pl.ds(g * 16, 16)] = vals
                return carry

            lax.fori_loop(0, _BPW // 16, extract, 0, unroll=False)
            pltpu.sync_copy(out_v, dst_out.at[:, pl.ds(base, _BPW)])

        gather_one(uid_hbm, ut2_hbm, u_out)
        gather_one(cid_hbm, ct2_hbm, c_out)

    return user_k


_kernel_cache = {}


def _get(name, maker):
    if name not in _kernel_cache:
        _kernel_cache[name] = maker()
    return _kernel_cache[name]


def _mlp_body(u_ref, c_ref, n_ref, w1u_ref, w1c_ref, w1n_ref, b1_ref,
              w2_ref, b2_ref, w3_ref, b3_ref, out_ref):
    h = (jnp.dot(w1u_ref[...], u_ref[...], preferred_element_type=jnp.float32)
         + jnp.dot(w1c_ref[...], c_ref[...], preferred_element_type=jnp.float32)
         + jnp.dot(w1n_ref[...], n_ref[...], preferred_element_type=jnp.float32)
         + b1_ref[...])
    h = jnp.maximum(h, 0.0)
    h = jnp.maximum(
        jnp.dot(w2_ref[...], h, preferred_element_type=jnp.float32) + b2_ref[...],
        0.0)
    out_ref[...] = (jnp.dot(w3_ref[...], h, preferred_element_type=jnp.float32)
                    + b3_ref[...])


def kernel(user_id, cat_id, numeric, user_table, cat_table, W1, b1, W2, b2, W3, b3):
    ut_t = user_table.T
    cpad = jnp.pad(cat_table.T, ((0, UD - CD), (0, _NCPAD - NCAT)))
    ut2, ct2 = _pack_table(ut_t, lax.slice(ut_t, (0, NU - _TAIL), (UD, NU)),
                           cpad)
    u_t, c_t = _get("user", _make_user_gather)(
        user_id.astype(jnp.int32), cat_id.astype(jnp.int32), ut2, ct2)
    w1c = jnp.pad(W1[:, UD:UD + CD], ((0, 0), (0, UD - CD)))
    out = pl.pallas_call(
        _mlp_body,
        out_shape=jax.ShapeDtypeStruct((1, B), jnp.float32),
    )(u_t, c_t, numeric.T, W1[:, :UD], w1c, W1[:, UD + CD:],
      b1.reshape(H1, 1), W2, b2.reshape(H2, 1), W3, b3.reshape(1, 1))
    return out.reshape(B)


# speedup vs baseline: 5.8358x; 1.0005x over previous
import functools

import jax
import jax.numpy as jnp
from jax import lax
from jax.experimental import pallas as pl
from jax.experimental.pallas import tpu as pltpu
from jax.experimental.pallas import tpu_sc as plsc

B = 16384
NU = 1000000
NCAT = 1000
UD = 16
CD = 8
ND = 8
H1 = 64
H2 = 32

_NC = 2
_NS = 16
_NW = _NC * _NS
_BPW = B // _NW
_CHUNK = 128
_NCH = _BPW // _CHUNK

_PACK = 128 // UD
_NROWS = NU // _PACK


_AC = 3072
_NFULL = 325
_NCHUNKS = _NFULL + 1
_LASTBASE = 996864
_TAIL = 64
_NPW_HI = -(-_NCHUNKS // _NW)
_HI_CUT = _NCHUNKS - (_NPW_HI - 1) * _NW
_NCPAD = 1024


def _make_pack():
    mesh = plsc.VectorSubcoreMesh(core_axis_name="c", subcore_axis_name="s")

    @functools.partial(
        pl.kernel,
        mesh=mesh,
        out_type=[jax.ShapeDtypeStruct((_NROWS, 128), jnp.float32),
                  jax.ShapeDtypeStruct((_NCPAD // _PACK, 128), jnp.float32)],
        scratch_types=[
            pltpu.VMEM((UD, _AC), jnp.float32),
            pltpu.VMEM((_AC // _PACK, 128), jnp.float32),
            pltpu.VMEM((UD, _TAIL), jnp.float32),
            pltpu.VMEM((_TAIL // _PACK, 128), jnp.float32),
            pltpu.SemaphoreType.DMA,
        ],
        compiler_params=pltpu.CompilerParams(needs_layout_passes=False),
    )
    def pack_k(ut_hbm, tail_hbm, cpad_hbm, ut2_out, ct2_out,
               in_v, out_v, tin_v, tout_v, sem):
        wid = lax.axis_index("s") * _NC + lax.axis_index("c")
        lane = lax.iota(jnp.int32, 16)
        rowbase = lax.shift_right_logical(lane, 3)
        cols = [(lane & 7) * UD + a for a in range(UD)]
        nmine = jnp.where(wid < _HI_CUT, _NPW_HI, _NPW_HI - 1)

        def transpose(src, dst, ngroups):
            def grp(g, carry):
                row = rowbase + 2 * g
                for a in range(UD):
                    vals = src[a, pl.ds(g * 16, 16)]
                    plsc.store_scatter(dst, [row, cols[a]], vals)
                return carry

            lax.fori_loop(0, ngroups, grp, 0)

        def chunk(i, carry):
            c = wid + i * _NW
            base_u = pl.multiple_of(
                jnp.where(c == _NCHUNKS - 1, _LASTBASE, c * _AC), 128)
            pltpu.sync_copy(ut_hbm.at[:, pl.ds(base_u, _AC)], in_v)
            transpose(in_v, out_v, _AC // 16)
            pltpu.sync_copy(
                out_v,
                ut2_out.at[pl.ds(pl.multiple_of(base_u // _PACK, 8),
                                 _AC // _PACK), :])
            return carry

        lax.fori_loop(0, nmine, chunk, 0)

        @pl.when(wid == 0)
        def _():
            pltpu.sync_copy(tail_hbm, tin_v)
            transpose(tin_v, tout_v, _TAIL // 16)
            pltpu.sync_copy(tout_v,
                            ut2_out.at[pl.ds(_NROWS - _TAIL // _PACK,
                                             _TAIL // _PACK), :])

        @pl.when(wid == _NW - 1)
        def _():
            pltpu.sync_copy(cpad_hbm, in_v.at[:, pl.ds(0, _NCPAD)])
            transpose(in_v, out_v, _NCPAD // 16)
            pltpu.sync_copy(out_v.at[pl.ds(0, _NCPAD // _PACK)], ct2_out)

    return pack_k


def _pack_table(ut_t, ut_tail, cpad):
    return _get("pack", _make_pack)(ut_t, ut_tail, cpad)


def _make_user_gather():
    mesh = plsc.VectorSubcoreMesh(core_axis_name="c", subcore_axis_name="s")

    @functools.partial(
        pl.kernel,
        mesh=mesh,
        out_type=[jax.ShapeDtypeStruct((UD, B), jnp.float32),
                  jax.ShapeDtypeStruct((UD, B), jnp.float32)],
        scratch_types=[
            pltpu.VMEM((_NCH, _CHUNK), jnp.int32),
            pltpu.VMEM((_NCH, _CHUNK), jnp.int32),
            pltpu.VMEM((_BPW, 128), jnp.float32),
            pltpu.VMEM((UD, _BPW), jnp.float32),
            pltpu.SemaphoreType.DMA,
        ],
        compiler_params=pltpu.CompilerParams(needs_layout_passes=False),
    )
    def user_k(uid_hbm, cid_hbm, ut2_hbm, ct2_hbm, u_out, c_out,
               idx_v, ridx_v, rows_v, out_v, sem):
        wid = lax.axis_index("s") * _NC + lax.axis_index("c")
        base = wid * _BPW
        lane = lax.iota(jnp.int32, 16)

        def gather_one(ids_hbm, tab_hbm, dst_out):
            for j in range(_NCH):
                pltpu.sync_copy(ids_hbm.at[pl.ds(base + j * _CHUNK, _CHUNK)],
                                idx_v.at[j])
            for j in range(_NCH):
                for t in range(_CHUNK // 16):
                    v = idx_v[j, pl.ds(t * 16, 16)]
                    ridx_v[j, pl.ds(t * 16, 16)] = lax.shift_right_logical(v, 3)
            copies = []
            for j in range(_NCH):
                copies.append(pltpu.async_copy(
                    tab_hbm.at[ridx_v.at[j]],
                    rows_v.at[pl.ds(j * _CHUNK, _CHUNK)], sem))
            for cp in copies:
                cp.wait()

            def extract(g, carry):
                j = g // (_CHUNK // 16)
                t = g % (_CHUNK // 16)
                ids = idx_v[j, pl.ds(t * 16, 16)]
                col0 = (ids & 7) * 16
                rows = g * 16 + lane
                for f in range(UD):
                    vals = plsc.load_gather(rows_v, [rows, col0 + f])
                    out_v[f, pl.ds(g * 16, 16)] = vals
                return carry

            lax.fori_loop(0, _BPW // 16, extract, 0, unroll=False)
            pltpu.sync_copy(out_v, dst_out.at[:, pl.ds(base, _BPW)])

        gather_one(uid_hbm, ut2_hbm, u_out)
        gather_one(cid_hbm, ct2_hbm, c_out)

    return user_k


_kernel_cache = {}


def _get(name, maker):
    if name not in _kernel_cache:
        _kernel_cache[name] = maker()
    return _kernel_cache[name]


def _mlp_body(u_ref, c_ref, n_ref, w1u_ref, w1c_ref, w1n_ref, b1_ref,
              w2_ref, b2_ref, w3_ref, b3_ref, out_ref):
    h = (jnp.dot(w1u_ref[...], u_ref[...], preferred_element_type=jnp.float32)
         + jnp.dot(w1c_ref[...], c_ref[...], preferred_element_type=jnp.float32)
         + jnp.dot(w1n_ref[...], n_ref[...], preferred_element_type=jnp.float32)
         + b1_ref[...])
    h = jnp.maximum(h, 0.0)
    h = jnp.maximum(
        jnp.dot(w2_ref[...], h, preferred_element_type=jnp.float32) + b2_ref[...],
        0.0)
    out_ref[...] = (jnp.dot(w3_ref[...], h, preferred_element_type=jnp.float32)
                    + b3_ref[...])


def kernel(user_id, cat_id, numeric, user_table, cat_table, W1, b1, W2, b2, W3, b3):
    ut_t = user_table.T
    cpad = jnp.pad(cat_table.T, ((0, UD - CD), (0, _NCPAD - NCAT)))
    ut2, ct2 = _pack_table(ut_t, lax.slice(ut_t, (0, NU - _TAIL), (UD, NU)),
                           cpad)
    u_t, c_t = _get("user", _make_user_gather)(
        user_id.astype(jnp.int32), cat_id.astype(jnp.int32), ut2, ct2)
    w1c = jnp.pad(W1[:, UD:UD + CD], ((0, 0), (0, UD - CD)))
    out = pl.pallas_call(
        _mlp_body,
        out_shape=jax.ShapeDtypeStruct((1, B), jnp.float32),
    )(u_t, c_t, numeric.T, W1[:, :UD], w1c, W1[:, UD + CD:],
      b1.reshape(H1, 1), W2, b2.reshape(H2, 1), W3, b3.reshape(1, 1))
    return out.reshape(B)


# speedup vs baseline: 7.6096x; 1.3040x over previous
import functools

import jax
import jax.numpy as jnp
from jax import lax
from jax.experimental import pallas as pl
from jax.experimental.pallas import tpu as pltpu
from jax.experimental.pallas import tpu_sc as plsc

B = 16384
NU = 1000000
NCAT = 1000
UD = 16
CD = 8
ND = 8
H1 = 64
H2 = 32

_NC = 2
_NS = 16
_NW = _NC * _NS
_BPW = B // _NW
_CHUNK = 128
_NCH = _BPW // _CHUNK

_PACK = 128 // UD
_NROWS = NU // _PACK


_AC = 1536
_NCHUNKS = 651
_TAIL = 64
_NPW_HI = -(-_NCHUNKS // _NW)
_HI_CUT = _NCHUNKS - (_NPW_HI - 1) * _NW
_NCPAD = 1024


def _make_pack():
    mesh = plsc.VectorSubcoreMesh(core_axis_name="c", subcore_axis_name="s")

    @functools.partial(
        pl.kernel,
        mesh=mesh,
        out_type=[jax.ShapeDtypeStruct((_NROWS, 128), jnp.float32),
                  jax.ShapeDtypeStruct((_NCPAD // _PACK, 128), jnp.float32)],
        scratch_types=[
            pltpu.VMEM((UD, _AC), jnp.float32),
            pltpu.VMEM((UD, _AC), jnp.float32),
            pltpu.VMEM((_AC // _PACK, 128), jnp.float32),
            pltpu.VMEM((_AC // _PACK, 128), jnp.float32),
            pltpu.VMEM((UD, _TAIL), jnp.float32),
            pltpu.VMEM((_TAIL // _PACK, 128), jnp.float32),
            pltpu.SemaphoreType.DMA,
            pltpu.SemaphoreType.DMA,
            pltpu.SemaphoreType.DMA,
            pltpu.SemaphoreType.DMA,
        ],
        compiler_params=pltpu.CompilerParams(needs_layout_passes=False),
    )
    def pack_k(ut_hbm, tail_hbm, cpad_hbm, ut2_out, ct2_out,
               in0, in1, out0, out1, tin_v, tout_v, si0, si1, so0, so1):
        wid = lax.axis_index("s") * _NC + lax.axis_index("c")
        lane = lax.iota(jnp.int32, 16)
        rowbase = lax.shift_right_logical(lane, 3)
        cols = [(lane & 7) * UD + a for a in range(UD)]
        nmine = jnp.where(wid < _HI_CUT, _NPW_HI, _NPW_HI - 1)

        def transpose(src, dst, ngroups):
            def grp(g, carry):
                row = rowbase + 2 * g
                for a in range(UD):
                    vals = src[a, pl.ds(g * 16, 16)]
                    plsc.store_scatter(dst, [row, cols[a]], vals)
                return carry

            lax.fori_loop(0, ngroups, grp, 0)

        def ubase(k):
            return pl.multiple_of((wid + k * _NW) * _AC, 128)

        def fire_in(k, buf, sem):
            pltpu.async_copy(ut_hbm.at[:, pl.ds(ubase(k), _AC)], buf, sem)

        def wait_in(buf, sem):
            pltpu.make_async_copy(ut_hbm.at[:, pl.ds(0, _AC)], buf, sem).wait()

        def fire_out(k, buf, sem):
            pltpu.async_copy(
                buf, ut2_out.at[pl.ds(pl.multiple_of(ubase(k) // _PACK, 8),
                                      _AC // _PACK), :], sem)

        def drain_out(buf, sem):
            pltpu.make_async_copy(buf, ut2_out.at[pl.ds(0, _AC // _PACK), :],
                                  sem).wait()

        fire_in(0, in0, si0)

        def pair(k2, carry):
            k0 = 2 * k2
            k1 = k0 + 1

            @pl.when(k1 < nmine)
            def _():
                fire_in(k1, in1, si1)

            @pl.when(k0 < nmine)
            def _():
                wait_in(in0, si0)

                @pl.when(k0 >= 2)
                def _():
                    drain_out(out0, so0)

                transpose(in0, out0, _AC // 16)
                fire_out(k0, out0, so0)

            @pl.when(k0 + 2 < nmine)
            def _():
                fire_in(k0 + 2, in0, si0)

            @pl.when(k1 < nmine)
            def _():
                wait_in(in1, si1)

                @pl.when(k1 >= 3)
                def _():
                    drain_out(out1, so1)

                transpose(in1, out1, _AC // 16)
                fire_out(k1, out1, so1)

            return carry

        lax.fori_loop(0, (nmine + 1) // 2, pair, 0)
        drain_out(out0, so0)
        drain_out(out1, so1)

        @pl.when(wid == 0)
        def _():
            pltpu.sync_copy(tail_hbm, tin_v)
            transpose(tin_v, tout_v, _TAIL // 16)
            pltpu.sync_copy(tout_v,
                            ut2_out.at[pl.ds(_NROWS - _TAIL // _PACK,
                                             _TAIL // _PACK), :])

        @pl.when(wid == _NW - 1)
        def _():
            pltpu.sync_copy(cpad_hbm, in0.at[:, pl.ds(0, _NCPAD)])
            transpose(in0, out0, _NCPAD // 16)
            pltpu.sync_copy(out0.at[pl.ds(0, _NCPAD // _PACK)], ct2_out)

    return pack_k


def _pack_table(ut_t, ut_tail, cpad):
    return _get("pack", _make_pack)(ut_t, ut_tail, cpad)


def _make_user_gather():
    mesh = plsc.VectorSubcoreMesh(core_axis_name="c", subcore_axis_name="s")

    @functools.partial(
        pl.kernel,
        mesh=mesh,
        out_type=[jax.ShapeDtypeStruct((UD, B), jnp.float32),
                  jax.ShapeDtypeStruct((UD, B), jnp.float32)],
        scratch_types=[
            pltpu.VMEM((_NCH, _CHUNK), jnp.int32),
            pltpu.VMEM((_NCH, _CHUNK), jnp.int32),
            pltpu.VMEM((_BPW, 128), jnp.float32),
            pltpu.VMEM((UD, _BPW), jnp.float32),
            pltpu.SemaphoreType.DMA,
        ],
        compiler_params=pltpu.CompilerParams(needs_layout_passes=False),
    )
    def user_k(uid_hbm, cid_hbm, ut2_hbm, ct2_hbm, u_out, c_out,
               idx_v, ridx_v, rows_v, out_v, sem):
        wid = lax.axis_index("s") * _NC + lax.axis_index("c")
        base = wid * _BPW
        lane = lax.iota(jnp.int32, 16)

        def gather_one(ids_hbm, tab_hbm, dst_out):
            for j in range(_NCH):
                pltpu.sync_copy(ids_hbm.at[pl.ds(base + j * _CHUNK, _CHUNK)],
                                idx_v.at[j])
            for j in range(_NCH):
                for t in range(_CHUNK // 16):
                    v = idx_v[j, pl.ds(t * 16, 16)]
                    ridx_v[j, pl.ds(t * 16, 16)] = lax.shift_right_logical(v, 3)
            copies = []
            for j in range(_NCH):
                copies.append(pltpu.async_copy(
                    tab_hbm.at[ridx_v.at[j]],
                    rows_v.at[pl.ds(j * _CHUNK, _CHUNK)], sem))
            for cp in copies:
                cp.wait()

            def extract(g, carry):
                j = g // (_CHUNK // 16)
                t = g % (_CHUNK // 16)
                ids = idx_v[j, pl.ds(t * 16, 16)]
                col0 = (ids & 7) * 16
                rows = g * 16 + lane
                for f in range(UD):
                    vals = plsc.load_gather(rows_v, [rows, col0 + f])
                    out_v[f, pl.ds(g * 16, 16)] = vals
                return carry

            lax.fori_loop(0, _BPW // 16, extract, 0, unroll=False)
            pltpu.sync_copy(out_v, dst_out.at[:, pl.ds(base, _BPW)])

        gather_one(uid_hbm, ut2_hbm, u_out)
        gather_one(cid_hbm, ct2_hbm, c_out)

    return user_k


_kernel_cache = {}


def _get(name, maker):
    if name not in _kernel_cache:
        _kernel_cache[name] = maker()
    return _kernel_cache[name]


def _mlp_body(u_ref, c_ref, n_ref, w1u_ref, w1c_ref, w1n_ref, b1_ref,
              w2_ref, b2_ref, w3_ref, b3_ref, out_ref):
    h = (jnp.dot(w1u_ref[...], u_ref[...], preferred_element_type=jnp.float32)
         + jnp.dot(w1c_ref[...], c_ref[...], preferred_element_type=jnp.float32)
         + jnp.dot(w1n_ref[...], n_ref[...], preferred_element_type=jnp.float32)
         + b1_ref[...])
    h = jnp.maximum(h, 0.0)
    h = jnp.maximum(
        jnp.dot(w2_ref[...], h, preferred_element_type=jnp.float32) + b2_ref[...],
        0.0)
    out_ref[...] = (jnp.dot(w3_ref[...], h, preferred_element_type=jnp.float32)
                    + b3_ref[...])


def kernel(user_id, cat_id, numeric, user_table, cat_table, W1, b1, W2, b2, W3, b3):
    ut_t = user_table.T
    cpad = jnp.pad(cat_table.T, ((0, UD - CD), (0, _NCPAD - NCAT)))
    ut2, ct2 = _pack_table(ut_t, lax.slice(ut_t, (0, NU - _TAIL), (UD, NU)),
                           cpad)
    u_t, c_t = _get("user", _make_user_gather)(
        user_id.astype(jnp.int32), cat_id.astype(jnp.int32), ut2, ct2)
    w1c = jnp.pad(W1[:, UD:UD + CD], ((0, 0), (0, UD - CD)))
    out = pl.pallas_call(
        _mlp_body,
        out_shape=jax.ShapeDtypeStruct((1, B), jnp.float32),
    )(u_t, c_t, numeric.T, W1[:, :UD], w1c, W1[:, UD + CD:],
      b1.reshape(H1, 1), W2, b2.reshape(H2, 1), W3, b3.reshape(1, 1))
    return out.reshape(B)
